# baseline (speedup 1.0000x reference)
import functools

import jax
import jax.numpy as jnp
from jax import lax
from jax.experimental import pallas as pl
from jax.experimental.pallas import tpu as pltpu
from jax.experimental.pallas import tpu_sc as plsc

N = 10000
E = 320000
H = 128
L = 4
LANES = 16
NC, NS = 2, 16
NW = NC * NS

ECHUNK = 128
NCH_PAD = 2560
CHPW = NCH_PAD // NW
E_PAD = NCH_PAD * ECHUNK
GB = 1
GBE = GB * ECHUNK
GRP = CHPW // GB
ACC_PAD = 16
SLAB = 624
TAIL = N - NS * SLAB
ZW = 78
OW = 104

NCHUNK = 80
NNCH = N // NCHUNK
EMB_BASE = NNCH // NW
EMB_REM = NNCH % NW


def _mesh():
    return plsc.VectorSubcoreMesh(
        core_axis_name="c", subcore_axis_name="s",
        num_cores=NC, num_subcores=NS)


def _zero_buf(buf, nrows):
    zeros = jnp.zeros((LANES,), jnp.float32)

    def row(r, carry):
        for j in range(H // LANES):
            buf[r, pl.ds(j * LANES, LANES)] = zeros
        return carry

    lax.fori_loop(0, nrows, row, 0)


@functools.partial(
    pl.kernel,
    out_type=jax.ShapeDtypeStruct((N, H), jnp.float32),
    mesh=_mesh(),
    scratch_types=[
        pltpu.VMEM((NCHUNK,), jnp.int32),
        pltpu.VMEM((NCHUNK,), jnp.int32),
        pltpu.VMEM((NCHUNK, H), jnp.float32),
        pltpu.VMEM((NCHUNK, H), jnp.float32),
        pltpu.SemaphoreType.DMA,
        pltpu.SemaphoreType.DMA,
    ],
)
def _embed(x0_hbm, x1_hbm, key_hbm, val_hbm, h_hbm,
           idx0, idx1, krows, vrows, sem0, sem1):
    c = lax.axis_index("c")
    s = lax.axis_index("s")
    w = s * NC + c
    nch = EMB_BASE + jnp.where(w < EMB_REM, 1, 0)
    c0 = EMB_BASE * w + jnp.minimum(w, EMB_REM)

    def body(k, carry):
        ch = c0 + k
        base = ch * NCHUNK
        pltpu.sync_copy(x0_hbm.at[pl.ds(base, NCHUNK)], idx0)
        pltpu.sync_copy(x1_hbm.at[pl.ds(base, NCHUNK)], idx1)
        cp0 = pltpu.async_copy(key_hbm.at[idx0], krows, sem0)
        cp1 = pltpu.async_copy(val_hbm.at[idx1], vrows, sem1)
        cp0.wait()
        cp1.wait()

        def row(r, cy):
            for j in range(H // LANES):
                sl = pl.ds(j * LANES, LANES)
                krows[r, sl] = krows[r, sl] + vrows[r, sl]
            return cy

        lax.fori_loop(0, NCHUNK, row, 0)
        pltpu.sync_copy(krows, h_hbm.at[pl.ds(base, NCHUNK)])
        return carry

    lax.fori_loop(0, nch, body, 0)


NBUF = 4


@functools.partial(
    pl.kernel,
    out_type=jax.ShapeDtypeStruct((NC, N, H), jnp.float32),
    mesh=_mesh(),
    scratch_types=[
        pltpu.VMEM((GRP * GBE,), jnp.int32),
        pltpu.VMEM((GRP * GBE,), jnp.int32),
        pltpu.VMEM((GBE, H), jnp.float32),
        pltpu.VMEM_SHARED((N + ACC_PAD, H), jnp.float32),
        pltpu.SemaphoreType.DMA,
    ],
)
def _edge_agg(h_hbm, src_hbm, dst_hbm, out_hbm,
              src_v, dst_v, rows, acc, gsem):
    c = lax.axis_index("c")
    s = lax.axis_index("s")
    w = s * NC + c
    c0 = GRP * w

    _zero_buf(rows, ZW)
    for k in range(SLAB // ZW):
        pltpu.sync_copy(rows.at[pl.ds(0, ZW)],
                        acc.at[pl.ds(s * SLAB + k * ZW, ZW)])

    @pl.when(s == NS - 1)
    def _():
        pltpu.sync_copy(rows.at[pl.ds(0, TAIL)],
                        acc.at[pl.ds(NS * SLAB, TAIL)])

    plsc.subcore_barrier()

    e0 = w * GRP * GBE
    pltpu.sync_copy(src_hbm.at[pl.ds(e0, GRP * GBE)], src_v)
    pltpu.sync_copy(dst_hbm.at[pl.ds(e0, GRP * GBE)], dst_v)

    pltpu.async_copy(h_hbm.at[src_v.at[pl.ds(0, GBE)]], rows, gsem)

    def body(k, carry):
        pltpu.make_async_copy(h_hbm.at[src_v.at[pl.ds(k * GBE, GBE)]],
                              rows, gsem).wait()
        pltpu.sync_copy(rows, acc.at[dst_v.at[pl.ds(k * GBE, GBE)]], add=True)

        @pl.when(k + 1 < GRP)
        def _():
            pltpu.async_copy(h_hbm.at[src_v.at[pl.ds((k + 1) * GBE, GBE)]],
                             rows, gsem)

        return carry

    lax.fori_loop(0, GRP, body, 0)
    plsc.subcore_barrier()

    for k in range(SLAB // OW):
        r0 = s * SLAB + k * OW
        pltpu.sync_copy(acc.at[pl.ds(r0, OW)], rows.at[pl.ds(0, OW)])
        pltpu.sync_copy(rows.at[pl.ds(0, OW)],
                        out_hbm.at[c, pl.ds(r0, OW)])

    @pl.when(s == NS - 1)
    def _():
        pltpu.sync_copy(acc.at[pl.ds(NS * SLAB, TAIL)],
                        rows.at[pl.ds(OW, TAIL)])
        pltpu.sync_copy(rows.at[pl.ds(OW, TAIL)],
                        out_hbm.at[c, pl.ds(NS * SLAB, TAIL)])


BLK = 1000


def _tc_layer_body(last, h_ref, p0_ref, p1_ref, w_ref, b_ref, g_ref, be_ref,
                   wo_ref, o_ref):
    agg = p0_ref[...] + p1_ref[...]
    z = jnp.dot(agg, w_ref[...], preferred_element_type=jnp.float32,
                precision=lax.Precision.HIGHEST) + b_ref[...]
    hn = h_ref[...] + jnp.maximum(z, 0.0)
    mu = jnp.mean(hn, axis=-1, keepdims=True)
    d = hn - mu
    var = jnp.mean(d * d, axis=-1, keepdims=True)
    ln = d * lax.rsqrt(var + 1e-5) * g_ref[...] + be_ref[...]
    if last:
        o_ref[...] = jnp.dot(ln, wo_ref[...], preferred_element_type=jnp.float32,
                             precision=lax.Precision.HIGHEST)
    else:
        o_ref[...] = ln


def _tc_layer(h, p0, p1, w, b, g, be, wo=None):
    last = wo is not None
    full = pl.BlockSpec((H, H), lambda i: (0, 0))
    vec = pl.BlockSpec((1, H), lambda i: (0, 0))
    row_blk = pl.BlockSpec((BLK, H), lambda i: (i, 0))
    in_specs = [row_blk, row_blk, row_blk, full, vec, vec, vec]
    args = [h, p0, p1, w, b.reshape(1, H), g.reshape(1, H), be.reshape(1, H)]
    if last:
        in_specs.append(full)
        args.append(wo)
    if last:
        def body(h_r, p0_r, p1_r, w_r, b_r, g_r, be_r, wo_r, o_r):
            _tc_layer_body(True, h_r, p0_r, p1_r, w_r, b_r, g_r, be_r, wo_r, o_r)
    else:
        def body(h_r, p0_r, p1_r, w_r, b_r, g_r, be_r, o_r):
            _tc_layer_body(False, h_r, p0_r, p1_r, w_r, b_r, g_r, be_r, None, o_r)
    return pl.pallas_call(
        body,
        grid=(N // BLK,),
        in_specs=in_specs,
        out_specs=row_blk,
        out_shape=jax.ShapeDtypeStruct((N, H), jnp.float32),
    )(*args)


def _tc_out(h, wo):
    def body(h_r, wo_r, o_r):
        o_r[...] = jnp.dot(h_r[...], wo_r[...],
                           preferred_element_type=jnp.float32,
                           precision=lax.Precision.HIGHEST)

    return pl.pallas_call(
        body,
        grid=(N // BLK,),
        in_specs=[pl.BlockSpec((BLK, H), lambda i: (i, 0)),
                  pl.BlockSpec((H, H), lambda i: (0, 0))],
        out_specs=pl.BlockSpec((BLK, H), lambda i: (i, 0)),
        out_shape=jax.ShapeDtypeStruct((N, H), jnp.float32),
    )(h, wo)


def kernel(x, edge_index, batch, key_table, val_table, Ws, bs, gammas, betas,
           W_out):
    del batch
    x0 = x[:, 0].astype(jnp.int32)
    x1 = x[:, 1].astype(jnp.int32)
    pad = E_PAD - E
    pad_src = (jnp.arange(pad, dtype=jnp.int32) * 37) % N
    pad_dst = N + (jnp.arange(pad, dtype=jnp.int32) % ACC_PAD)
    src = jnp.concatenate([edge_index[0].astype(jnp.int32), pad_src]
                          )
    dst = jnp.concatenate([edge_index[1].astype(jnp.int32), pad_dst])

    h = _embed(x0, x1, key_table, val_table)

    def step(hc, layer_params):
        w, b, g, be = layer_params
        parts = _edge_agg(hc, src, dst)
        return _tc_layer(hc, parts[0], parts[1], w, b, g, be, None), None

    h, _ = lax.scan(step, h, (Ws[:L - 1], bs[:L - 1],
                              gammas[:L - 1], betas[:L - 1]))
    parts = _edge_agg(h, src, dst)
    return _tc_layer(h, parts[0], parts[1], Ws[L - 1], bs[L - 1],
                     gammas[L - 1], betas[L - 1], W_out)


# speedup vs baseline: 1.0035x; 1.0035x over previous
import functools

import jax
import jax.numpy as jnp
from jax import lax
from jax.experimental import pallas as pl
from jax.experimental.pallas import tpu as pltpu
from jax.experimental.pallas import tpu_sc as plsc

N = 10000
E = 320000
H = 128
L = 4
LANES = 16
NC, NS = 2, 16
NW = NC * NS

ECHUNK = 128
NCH_PAD = 2560
CHPW = NCH_PAD // NW
E_PAD = NCH_PAD * ECHUNK
GB = 1
GBE = GB * ECHUNK
GRP = CHPW // GB
ACC_PAD = 16
SLAB = 624
TAIL = N - NS * SLAB
ZW = 78
OW = 104

NCHUNK = 80
NNCH = N // NCHUNK
EMB_BASE = NNCH // NW
EMB_REM = NNCH % NW


def _mesh():
    return plsc.VectorSubcoreMesh(
        core_axis_name="c", subcore_axis_name="s",
        num_cores=NC, num_subcores=NS)


def _zero_buf(buf, nrows):
    zeros = jnp.zeros((LANES,), jnp.float32)

    def row(r, carry):
        for j in range(H // LANES):
            buf[r, pl.ds(j * LANES, LANES)] = zeros
        return carry

    lax.fori_loop(0, nrows, row, 0)


@functools.partial(
    pl.kernel,
    out_type=jax.ShapeDtypeStruct((N, H), jnp.float32),
    mesh=_mesh(),
    scratch_types=[
        pltpu.VMEM((NCHUNK,), jnp.int32),
        pltpu.VMEM((NCHUNK,), jnp.int32),
        pltpu.VMEM((NCHUNK, H), jnp.float32),
        pltpu.VMEM((NCHUNK, H), jnp.float32),
        pltpu.SemaphoreType.DMA,
        pltpu.SemaphoreType.DMA,
    ],
)
def _embed(x0_hbm, x1_hbm, key_hbm, val_hbm, h_hbm,
           idx0, idx1, krows, vrows, sem0, sem1):
    c = lax.axis_index("c")
    s = lax.axis_index("s")
    w = s * NC + c
    nch = EMB_BASE + jnp.where(w < EMB_REM, 1, 0)
    c0 = EMB_BASE * w + jnp.minimum(w, EMB_REM)

    def body(k, carry):
        ch = c0 + k
        base = ch * NCHUNK
        pltpu.sync_copy(x0_hbm.at[pl.ds(base, NCHUNK)], idx0)
        pltpu.sync_copy(x1_hbm.at[pl.ds(base, NCHUNK)], idx1)
        cp0 = pltpu.async_copy(key_hbm.at[idx0], krows, sem0)
        cp1 = pltpu.async_copy(val_hbm.at[idx1], vrows, sem1)
        cp0.wait()
        cp1.wait()

        def row(r, cy):
            for j in range(H // LANES):
                sl = pl.ds(j * LANES, LANES)
                krows[r, sl] = krows[r, sl] + vrows[r, sl]
            return cy

        lax.fori_loop(0, NCHUNK, row, 0)
        pltpu.sync_copy(krows, h_hbm.at[pl.ds(base, NCHUNK)])
        return carry

    lax.fori_loop(0, nch, body, 0)


NBUF = 4


@functools.partial(
    pl.kernel,
    out_type=jax.ShapeDtypeStruct((NC, N, H), jnp.float32),
    mesh=_mesh(),
    scratch_types=[
        pltpu.VMEM((GRP * GBE,), jnp.int32),
        pltpu.VMEM((GRP * GBE,), jnp.int32),
        pltpu.VMEM((GBE, H), jnp.float32),
        pltpu.VMEM_SHARED((N + ACC_PAD, H), jnp.float32),
        pltpu.SemaphoreType.DMA,
    ],
)
def _edge_agg(h_hbm, src_hbm, dst_hbm, out_hbm,
              src_v, dst_v, rows, acc, gsem):
    c = lax.axis_index("c")
    s = lax.axis_index("s")
    w = s * NC + c
    c0 = GRP * w

    _zero_buf(rows, ZW)
    for k in range(SLAB // ZW):
        pltpu.sync_copy(rows.at[pl.ds(0, ZW)],
                        acc.at[pl.ds(s * SLAB + k * ZW, ZW)])

    @pl.when(s == NS - 1)
    def _():
        pltpu.sync_copy(rows.at[pl.ds(0, TAIL)],
                        acc.at[pl.ds(NS * SLAB, TAIL)])

    plsc.subcore_barrier()

    e0 = w * GRP * GBE
    pltpu.sync_copy(src_hbm.at[pl.ds(e0, GRP * GBE)], src_v)
    pltpu.sync_copy(dst_hbm.at[pl.ds(e0, GRP * GBE)], dst_v)

    pltpu.async_copy(h_hbm.at[src_v.at[pl.ds(0, GBE)]], rows, gsem)

    def body(k, carry):
        pltpu.make_async_copy(h_hbm.at[src_v.at[pl.ds(k * GBE, GBE)]],
                              rows, gsem).wait()
        pltpu.sync_copy(rows, acc.at[dst_v.at[pl.ds(k * GBE, GBE)]], add=True)

        @pl.when(k + 1 < GRP)
        def _():
            pltpu.async_copy(h_hbm.at[src_v.at[pl.ds((k + 1) * GBE, GBE)]],
                             rows, gsem)

        return carry

    lax.fori_loop(0, GRP, body, 0)
    plsc.subcore_barrier()

    for k in range(SLAB // OW):
        r0 = s * SLAB + k * OW
        pltpu.sync_copy(acc.at[pl.ds(r0, OW)], rows.at[pl.ds(0, OW)])
        pltpu.sync_copy(rows.at[pl.ds(0, OW)],
                        out_hbm.at[c, pl.ds(r0, OW)])

    @pl.when(s == NS - 1)
    def _():
        pltpu.sync_copy(acc.at[pl.ds(NS * SLAB, TAIL)],
                        rows.at[pl.ds(OW, TAIL)])
        pltpu.sync_copy(rows.at[pl.ds(OW, TAIL)],
                        out_hbm.at[c, pl.ds(NS * SLAB, TAIL)])


BLK = 1000


def _tc_layer_body(last, h_ref, p0_ref, p1_ref, w_ref, b_ref, g_ref, be_ref,
                   wo_ref, o_ref):
    agg = p0_ref[...] + p1_ref[...]
    z = jnp.dot(agg, w_ref[...], preferred_element_type=jnp.float32,
                precision=lax.Precision.HIGHEST) + b_ref[...]
    hn = h_ref[...] + jnp.maximum(z, 0.0)
    mu = jnp.mean(hn, axis=-1, keepdims=True)
    d = hn - mu
    var = jnp.mean(d * d, axis=-1, keepdims=True)
    ln = d * lax.rsqrt(var + 1e-5) * g_ref[...] + be_ref[...]
    if last:
        o_ref[...] = jnp.dot(ln, wo_ref[...], preferred_element_type=jnp.float32,
                             precision=lax.Precision.HIGHEST)
    else:
        o_ref[...] = ln


def _tc_layer(h, p0, p1, w, b, g, be, wo=None):
    last = wo is not None
    full = pl.BlockSpec((H, H), lambda i: (0, 0))
    vec = pl.BlockSpec((1, H), lambda i: (0, 0))
    row_blk = pl.BlockSpec((BLK, H), lambda i: (i, 0))
    in_specs = [row_blk, row_blk, row_blk, full, vec, vec, vec]
    args = [h, p0, p1, w, b.reshape(1, H), g.reshape(1, H), be.reshape(1, H)]
    if last:
        in_specs.append(full)
        args.append(wo)
    if last:
        def body(h_r, p0_r, p1_r, w_r, b_r, g_r, be_r, wo_r, o_r):
            _tc_layer_body(True, h_r, p0_r, p1_r, w_r, b_r, g_r, be_r, wo_r, o_r)
    else:
        def body(h_r, p0_r, p1_r, w_r, b_r, g_r, be_r, o_r):
            _tc_layer_body(False, h_r, p0_r, p1_r, w_r, b_r, g_r, be_r, None, o_r)
    return pl.pallas_call(
        body,
        grid=(N // BLK,),
        in_specs=in_specs,
        out_specs=row_blk,
        out_shape=jax.ShapeDtypeStruct((N, H), jnp.float32),
    )(*args)


def _tc_out(h, wo):
    def body(h_r, wo_r, o_r):
        o_r[...] = jnp.dot(h_r[...], wo_r[...],
                           preferred_element_type=jnp.float32,
                           precision=lax.Precision.HIGHEST)

    return pl.pallas_call(
        body,
        grid=(N // BLK,),
        in_specs=[pl.BlockSpec((BLK, H), lambda i: (i, 0)),
                  pl.BlockSpec((H, H), lambda i: (0, 0))],
        out_specs=pl.BlockSpec((BLK, H), lambda i: (i, 0)),
        out_shape=jax.ShapeDtypeStruct((N, H), jnp.float32),
    )(h, wo)


def kernel(x, edge_index, batch, key_table, val_table, Ws, bs, gammas, betas,
           W_out):
    del batch
    x0 = x[:, 0].astype(jnp.int32)
    x1 = x[:, 1].astype(jnp.int32)
    pad = E_PAD - E
    pad_src = (jnp.arange(pad, dtype=jnp.int32) * 37) % N
    pad_dst = N + (jnp.arange(pad, dtype=jnp.int32) % ACC_PAD)
    src = jnp.concatenate([edge_index[0].astype(jnp.int32), pad_src]
                          )
    dst = jnp.concatenate([edge_index[1].astype(jnp.int32), pad_dst])

    h = _embed(x0, x1, key_table, val_table)

    def step(hc, layer_params):
        w, b, g, be = layer_params
        parts = _edge_agg(hc, src, dst)
        return _tc_layer(hc, parts[0], parts[1], w, b, g, be, None), None

    h, _ = lax.scan(step, h, (Ws, bs, gammas, betas))
    return _tc_out(h, W_out)


# speedup vs baseline: 1.3917x; 1.3868x over previous
import functools

import jax
import jax.numpy as jnp
from jax import lax
from jax.experimental import pallas as pl
from jax.experimental.pallas import tpu as pltpu
from jax.experimental.pallas import tpu_sc as plsc

N = 10000
E = 320000
H = 128
L = 4
LANES = 16
NC, NS = 2, 16
NW = NC * NS

SLOT = 128
CPE = 112
NCH_PAD = 2944
GRP = NCH_PAD // NW
GRP_H = GRP // 2
GBE = SLOT
E_PAD = NCH_PAD * CPE
ACC_PAD = 8
SLAB = 624
TAIL = N - NS * SLAB
ZW = 78
OW = 104

NCHUNK = 80
NNCH = N // NCHUNK
EMB_BASE = NNCH // NW
EMB_REM = NNCH % NW


def _mesh():
    return plsc.VectorSubcoreMesh(
        core_axis_name="c", subcore_axis_name="s",
        num_cores=NC, num_subcores=NS)


def _zero_buf(buf, nrows):
    zeros = jnp.zeros((LANES,), jnp.float32)

    def row(r, carry):
        for j in range(H // LANES):
            buf[r, pl.ds(j * LANES, LANES)] = zeros
        return carry

    lax.fori_loop(0, nrows, row, 0)


@functools.partial(
    pl.kernel,
    out_type=jax.ShapeDtypeStruct((N, H), jnp.float32),
    mesh=_mesh(),
    scratch_types=[
        pltpu.VMEM((NCHUNK,), jnp.int32),
        pltpu.VMEM((NCHUNK,), jnp.int32),
        pltpu.VMEM((NCHUNK, H), jnp.float32),
        pltpu.VMEM((NCHUNK, H), jnp.float32),
        pltpu.SemaphoreType.DMA,
        pltpu.SemaphoreType.DMA,
    ],
)
def _embed(x0_hbm, x1_hbm, key_hbm, val_hbm, h_hbm,
           idx0, idx1, krows, vrows, sem0, sem1):
    c = lax.axis_index("c")
    s = lax.axis_index("s")
    w = s * NC + c
    nch = EMB_BASE + jnp.where(w < EMB_REM, 1, 0)
    c0 = EMB_BASE * w + jnp.minimum(w, EMB_REM)

    def body(k, carry):
        ch = c0 + k
        base = ch * NCHUNK
        pltpu.sync_copy(x0_hbm.at[pl.ds(base, NCHUNK)], idx0)
        pltpu.sync_copy(x1_hbm.at[pl.ds(base, NCHUNK)], idx1)
        cp0 = pltpu.async_copy(key_hbm.at[idx0], krows, sem0)
        cp1 = pltpu.async_copy(val_hbm.at[idx1], vrows, sem1)
        cp0.wait()
        cp1.wait()

        def row(r, cy):
            for j in range(H // LANES):
                sl = pl.ds(j * LANES, LANES)
                krows[r, sl] = krows[r, sl] + vrows[r, sl]
            return cy

        lax.fori_loop(0, NCHUNK, row, 0)
        pltpu.sync_copy(krows, h_hbm.at[pl.ds(base, NCHUNK)])
        return carry

    lax.fori_loop(0, nch, body, 0)


NBUF = 4


@functools.partial(
    pl.kernel,
    out_type=jax.ShapeDtypeStruct((NC, N, H), jnp.float32),
    mesh=_mesh(),
    scratch_types=[
        pltpu.VMEM((GRP_H * SLOT,), jnp.int32),
        pltpu.VMEM((GRP_H * SLOT,), jnp.int32),
        pltpu.VMEM((CPE, H), jnp.float32),
        pltpu.VMEM((CPE, H), jnp.float32),
        pltpu.VMEM_SHARED((N + ACC_PAD, H), jnp.float32),
        pltpu.SemaphoreType.DMA,
        pltpu.SemaphoreType.DMA,
    ],
)
def _edge_agg(h_hbm, src_hbm, dst_hbm, out_hbm,
              src_v, dst_v, rows, rows2, acc, gsem, gsem2):
    c = lax.axis_index("c")
    s = lax.axis_index("s")
    w = s * NC + c
    c0 = GRP * w

    _zero_buf(rows, ZW)
    for k in range(SLAB // ZW):
        pltpu.sync_copy(rows.at[pl.ds(0, ZW)],
                        acc.at[pl.ds(s * SLAB + k * ZW, ZW)])

    @pl.when(s == NS - 1)
    def _():
        pltpu.sync_copy(rows.at[pl.ds(0, TAIL)],
                        acc.at[pl.ds(NS * SLAB, TAIL)])

    plsc.subcore_barrier()

    e0 = w * GRP * SLOT
    for h2 in range(2):
        s0 = e0 + h2 * GRP_H * SLOT
        pltpu.sync_copy(src_hbm.at[pl.ds(s0, GRP_H * SLOT)], src_v)
        pltpu.sync_copy(dst_hbm.at[pl.ds(s0, GRP_H * SLOT)], dst_v)
        pltpu.async_copy(h_hbm.at[src_v.at[pl.ds(0, CPE)]], rows, gsem)

        def body(j, carry):
            k = 2 * j
            pltpu.make_async_copy(h_hbm.at[src_v.at[pl.ds(k * SLOT, CPE)]],
                                  rows, gsem).wait()
            pltpu.async_copy(
                h_hbm.at[src_v.at[pl.ds((k + 1) * SLOT, CPE)]], rows2, gsem2)
            pltpu.sync_copy(rows, acc.at[dst_v.at[pl.ds(k * SLOT, CPE)]],
                            add=True)

            @pl.when(k + 2 < GRP_H)
            def _():
                pltpu.async_copy(
                    h_hbm.at[src_v.at[pl.ds((k + 2) * SLOT, CPE)]], rows,
                    gsem)

            pltpu.make_async_copy(
                h_hbm.at[src_v.at[pl.ds((k + 1) * SLOT, CPE)]],
                rows2, gsem2).wait()
            pltpu.sync_copy(rows2, acc.at[dst_v.at[pl.ds((k + 1) * SLOT, CPE)]],
                            add=True)
            return carry

        lax.fori_loop(0, GRP_H // 2, body, 0)

    plsc.subcore_barrier()

    for k in range(SLAB // OW):
        r0 = s * SLAB + k * OW
        pltpu.sync_copy(acc.at[pl.ds(r0, OW)], rows.at[pl.ds(0, OW)])
        pltpu.sync_copy(rows.at[pl.ds(0, OW)],
                        out_hbm.at[c, pl.ds(r0, OW)])

    @pl.when(s == NS - 1)
    def _():
        pltpu.sync_copy(acc.at[pl.ds(NS * SLAB, TAIL)],
                        rows2.at[pl.ds(0, TAIL)])
        pltpu.sync_copy(rows2.at[pl.ds(0, TAIL)],
                        out_hbm.at[c, pl.ds(NS * SLAB, TAIL)])


BLK = 1000


def _tc_layer_body(last, h_ref, p0_ref, p1_ref, w_ref, b_ref, g_ref, be_ref,
                   wo_ref, o_ref):
    agg = p0_ref[...] + p1_ref[...]
    z = jnp.dot(agg, w_ref[...], preferred_element_type=jnp.float32,
                precision=lax.Precision.HIGHEST) + b_ref[...]
    hn = h_ref[...] + jnp.maximum(z, 0.0)
    mu = jnp.mean(hn, axis=-1, keepdims=True)
    d = hn - mu
    var = jnp.mean(d * d, axis=-1, keepdims=True)
    ln = d * lax.rsqrt(var + 1e-5) * g_ref[...] + be_ref[...]
    if last:
        o_ref[...] = jnp.dot(ln, wo_ref[...], preferred_element_type=jnp.float32,
                             precision=lax.Precision.HIGHEST)
    else:
        o_ref[...] = ln


def _tc_layer(h, p0, p1, w, b, g, be, wo=None):
    last = wo is not None
    full = pl.BlockSpec((H, H), lambda i: (0, 0))
    vec = pl.BlockSpec((1, H), lambda i: (0, 0))
    row_blk = pl.BlockSpec((BLK, H), lambda i: (i, 0))
    in_specs = [row_blk, row_blk, row_blk, full, vec, vec, vec]
    args = [h, p0, p1, w, b.reshape(1, H), g.reshape(1, H), be.reshape(1, H)]
    if last:
        in_specs.append(full)
        args.append(wo)
    if last:
        def body(h_r, p0_r, p1_r, w_r, b_r, g_r, be_r, wo_r, o_r):
            _tc_layer_body(True, h_r, p0_r, p1_r, w_r, b_r, g_r, be_r, wo_r, o_r)
    else:
        def body(h_r, p0_r, p1_r, w_r, b_r, g_r, be_r, o_r):
            _tc_layer_body(False, h_r, p0_r, p1_r, w_r, b_r, g_r, be_r, None, o_r)
    return pl.pallas_call(
        body,
        grid=(N // BLK,),
        in_specs=in_specs,
        out_specs=row_blk,
        out_shape=jax.ShapeDtypeStruct((N, H), jnp.float32),
    )(*args)


def _tc_out(h, wo):
    def body(h_r, wo_r, o_r):
        o_r[...] = jnp.dot(h_r[...], wo_r[...],
                           preferred_element_type=jnp.float32,
                           precision=lax.Precision.HIGHEST)

    return pl.pallas_call(
        body,
        grid=(N // BLK,),
        in_specs=[pl.BlockSpec((BLK, H), lambda i: (i, 0)),
                  pl.BlockSpec((H, H), lambda i: (0, 0))],
        out_specs=pl.BlockSpec((BLK, H), lambda i: (i, 0)),
        out_shape=jax.ShapeDtypeStruct((N, H), jnp.float32),
    )(h, wo)


def kernel(x, edge_index, batch, key_table, val_table, Ws, bs, gammas, betas,
           W_out):
    del batch
    x0 = x[:, 0].astype(jnp.int32)
    x1 = x[:, 1].astype(jnp.int32)
    pad = E_PAD - E
    pad_src = (jnp.arange(pad, dtype=jnp.int32) * 37) % N
    pad_dst = N + (jnp.arange(pad, dtype=jnp.int32) % ACC_PAD)

    def slot(a, fill):
        a = jnp.concatenate([a.astype(jnp.int32), fill]).reshape(NCH_PAD, CPE)
        return jnp.pad(a, ((0, 0), (0, SLOT - CPE))).reshape(-1)

    src = slot(edge_index[0], pad_src)
    dst = slot(edge_index[1], pad_dst)

    h = _embed(x0, x1, key_table, val_table)

    def step(hc, layer_params):
        w, b, g, be = layer_params
        parts = _edge_agg(hc, src, dst)
        return _tc_layer(hc, parts[0], parts[1], w, b, g, be, None), None

    h, _ = lax.scan(step, h, (Ws, bs, gammas, betas))
    return _tc_out(h, W_out)


# speedup vs baseline: 1.4410x; 1.0354x over previous
import functools

import jax
import jax.numpy as jnp
from jax import lax
from jax.experimental import pallas as pl
from jax.experimental.pallas import tpu as pltpu
from jax.experimental.pallas import tpu_sc as plsc

N = 10000
E = 320000
H = 128
L = 4
LANES = 16
NC, NS = 2, 16
NW = NC * NS

SLOT = 128
CPE = 112
NCH_PAD = 2944
GRP = NCH_PAD // NW
GRP_H = GRP // 2
GBE = SLOT
E_PAD = NCH_PAD * CPE
ACC_PAD = 8
SLAB = 624
TAIL = N - NS * SLAB
ZW = 78
OW = 104

NCHUNK = 80
NNCH = N // NCHUNK
EMB_BASE = NNCH // NW
EMB_REM = NNCH % NW


def _mesh():
    return plsc.VectorSubcoreMesh(
        core_axis_name="c", subcore_axis_name="s",
        num_cores=NC, num_subcores=NS)


def _zero_buf(buf, nrows):
    zeros = jnp.zeros((LANES,), jnp.float32)

    def row(r, carry):
        for j in range(H // LANES):
            buf[r, pl.ds(j * LANES, LANES)] = zeros
        return carry

    lax.fori_loop(0, nrows, row, 0)


@functools.partial(
    pl.kernel,
    out_type=jax.ShapeDtypeStruct((N, H), jnp.float32),
    mesh=_mesh(),
    scratch_types=[
        pltpu.VMEM((NCHUNK,), jnp.int32),
        pltpu.VMEM((NCHUNK,), jnp.int32),
        pltpu.VMEM((NCHUNK, H), jnp.float32),
        pltpu.VMEM((NCHUNK, H), jnp.float32),
        pltpu.SemaphoreType.DMA,
        pltpu.SemaphoreType.DMA,
    ],
)
def _embed(x0_hbm, x1_hbm, key_hbm, val_hbm, h_hbm,
           idx0, idx1, krows, vrows, sem0, sem1):
    c = lax.axis_index("c")
    s = lax.axis_index("s")
    w = s * NC + c
    nch = EMB_BASE + jnp.where(w < EMB_REM, 1, 0)
    c0 = EMB_BASE * w + jnp.minimum(w, EMB_REM)

    def body(k, carry):
        ch = c0 + k
        base = ch * NCHUNK
        pltpu.sync_copy(x0_hbm.at[pl.ds(base, NCHUNK)], idx0)
        pltpu.sync_copy(x1_hbm.at[pl.ds(base, NCHUNK)], idx1)
        cp0 = pltpu.async_copy(key_hbm.at[idx0], krows, sem0)
        cp1 = pltpu.async_copy(val_hbm.at[idx1], vrows, sem1)
        cp0.wait()
        cp1.wait()

        def row(r, cy):
            for j in range(H // LANES):
                sl = pl.ds(j * LANES, LANES)
                krows[r, sl] = krows[r, sl] + vrows[r, sl]
            return cy

        lax.fori_loop(0, NCHUNK, row, 0)
        pltpu.sync_copy(krows, h_hbm.at[pl.ds(base, NCHUNK)])
        return carry

    lax.fori_loop(0, nch, body, 0)


NBUF = 4


@functools.partial(
    pl.kernel,
    out_type=jax.ShapeDtypeStruct((NC, N, H), jnp.float32),
    mesh=_mesh(),
    scratch_types=[
        pltpu.VMEM((GRP_H * SLOT,), jnp.int32),
        pltpu.VMEM((GRP_H * SLOT,), jnp.int32),
        pltpu.VMEM((CPE, H), jnp.float32),
        pltpu.VMEM((CPE, H), jnp.float32),
        pltpu.VMEM_SHARED((N + ACC_PAD, H), jnp.float32),
        pltpu.SemaphoreType.DMA,
        pltpu.SemaphoreType.DMA,
    ],
)
def _edge_agg(h_hbm, src_hbm, dst_hbm, out_hbm,
              src_v, dst_v, rows, rows2, acc, gsem, gsem2):
    c = lax.axis_index("c")
    s = lax.axis_index("s")
    w = s * NC + c
    c0 = GRP * w

    _zero_buf(rows, ZW)
    for k in range(SLAB // ZW):
        pltpu.sync_copy(rows.at[pl.ds(0, ZW)],
                        acc.at[pl.ds(s * SLAB + k * ZW, ZW)])

    @pl.when(s == NS - 1)
    def _():
        pltpu.sync_copy(rows.at[pl.ds(0, TAIL)],
                        acc.at[pl.ds(NS * SLAB, TAIL)])

    plsc.subcore_barrier()

    e0 = w * GRP * SLOT
    for h2 in range(2):
        s0 = e0 + h2 * GRP_H * SLOT
        pltpu.sync_copy(src_hbm.at[pl.ds(s0, GRP_H * SLOT)], src_v)
        pltpu.sync_copy(dst_hbm.at[pl.ds(s0, GRP_H * SLOT)], dst_v)
        pltpu.async_copy(h_hbm.at[src_v.at[pl.ds(0, CPE)]], rows, gsem)

        def body(j, carry):
            k = 2 * j
            pltpu.make_async_copy(h_hbm.at[src_v.at[pl.ds(k * SLOT, CPE)]],
                                  rows, gsem).wait()
            pltpu.async_copy(
                h_hbm.at[src_v.at[pl.ds((k + 1) * SLOT, CPE)]], rows2, gsem2)
            pltpu.sync_copy(rows, acc.at[dst_v.at[pl.ds(k * SLOT, CPE)]],
                            add=True)

            @pl.when(k + 2 < GRP_H)
            def _():
                pltpu.async_copy(
                    h_hbm.at[src_v.at[pl.ds((k + 2) * SLOT, CPE)]], rows,
                    gsem)

            pltpu.make_async_copy(
                h_hbm.at[src_v.at[pl.ds((k + 1) * SLOT, CPE)]],
                rows2, gsem2).wait()
            pltpu.sync_copy(rows2, acc.at[dst_v.at[pl.ds((k + 1) * SLOT, CPE)]],
                            add=True)
            return carry

        lax.fori_loop(0, GRP_H // 2, body, 0)

    plsc.subcore_barrier()

    for k in range(SLAB // OW):
        r0 = s * SLAB + k * OW
        pltpu.sync_copy(acc.at[pl.ds(r0, OW)], rows.at[pl.ds(0, OW)])
        pltpu.sync_copy(rows.at[pl.ds(0, OW)],
                        out_hbm.at[c, pl.ds(r0, OW)])

    @pl.when(s == NS - 1)
    def _():
        pltpu.sync_copy(acc.at[pl.ds(NS * SLAB, TAIL)],
                        rows2.at[pl.ds(0, TAIL)])
        pltpu.sync_copy(rows2.at[pl.ds(0, TAIL)],
                        out_hbm.at[c, pl.ds(NS * SLAB, TAIL)])


BLK = 2000


def _tc_layer_body(last, h_ref, p0_ref, p1_ref, w_ref, b_ref, g_ref, be_ref,
                   wo_ref, o_ref):
    agg = p0_ref[...] + p1_ref[...]
    z = jnp.dot(agg, w_ref[...], preferred_element_type=jnp.float32,
                precision=lax.Precision.HIGHEST) + b_ref[...]
    hn = h_ref[...] + jnp.maximum(z, 0.0)
    mu = jnp.mean(hn, axis=-1, keepdims=True)
    d = hn - mu
    var = jnp.mean(d * d, axis=-1, keepdims=True)
    ln = d * lax.rsqrt(var + 1e-5) * g_ref[...] + be_ref[...]
    if last:
        o_ref[...] = jnp.dot(ln, wo_ref[...], preferred_element_type=jnp.float32,
                             precision=lax.Precision.HIGHEST)
    else:
        o_ref[...] = ln


def _tc_layer(h, p0, p1, w, b, g, be, wo=None):
    last = wo is not None
    full = pl.BlockSpec((H, H), lambda i: (0, 0))
    vec = pl.BlockSpec((1, H), lambda i: (0, 0))
    row_blk = pl.BlockSpec((BLK, H), lambda i: (i, 0))
    in_specs = [row_blk, row_blk, row_blk, full, vec, vec, vec]
    args = [h, p0, p1, w, b.reshape(1, H), g.reshape(1, H), be.reshape(1, H)]
    if last:
        in_specs.append(full)
        args.append(wo)
    if last:
        def body(h_r, p0_r, p1_r, w_r, b_r, g_r, be_r, wo_r, o_r):
            _tc_layer_body(True, h_r, p0_r, p1_r, w_r, b_r, g_r, be_r, wo_r, o_r)
    else:
        def body(h_r, p0_r, p1_r, w_r, b_r, g_r, be_r, o_r):
            _tc_layer_body(False, h_r, p0_r, p1_r, w_r, b_r, g_r, be_r, None, o_r)
    return pl.pallas_call(
        body,
        grid=(N // BLK,),
        in_specs=in_specs,
        out_specs=row_blk,
        out_shape=jax.ShapeDtypeStruct((N, H), jnp.float32),
    )(*args)


def _tc_out(h, wo):
    def body(h_r, wo_r, o_r):
        o_r[...] = jnp.dot(h_r[...], wo_r[...],
                           preferred_element_type=jnp.float32,
                           precision=lax.Precision.HIGHEST)

    return pl.pallas_call(
        body,
        grid=(N // BLK,),
        in_specs=[pl.BlockSpec((BLK, H), lambda i: (i, 0)),
                  pl.BlockSpec((H, H), lambda i: (0, 0))],
        out_specs=pl.BlockSpec((BLK, H), lambda i: (i, 0)),
        out_shape=jax.ShapeDtypeStruct((N, H), jnp.float32),
    )(h, wo)


def kernel(x, edge_index, batch, key_table, val_table, Ws, bs, gammas, betas,
           W_out):
    del batch
    x0 = x[:, 0].astype(jnp.int32)
    x1 = x[:, 1].astype(jnp.int32)
    pad = E_PAD - E
    pad_src = (jnp.arange(pad, dtype=jnp.int32) * 37) % N
    pad_dst = N + (jnp.arange(pad, dtype=jnp.int32) % ACC_PAD)

    def slot(a, fill):
        a = jnp.concatenate([a.astype(jnp.int32), fill]).reshape(NCH_PAD, CPE)
        return jnp.pad(a, ((0, 0), (0, SLOT - CPE))).reshape(-1)

    src = slot(edge_index[0], pad_src)
    dst = slot(edge_index[1], pad_dst)

    h = _embed(x0, x1, key_table, val_table)

    def step(hc, layer_params):
        w, b, g, be = layer_params
        parts = _edge_agg(hc, src, dst)
        return _tc_layer(hc, parts[0], parts[1], w, b, g, be, None), None

    h, _ = lax.scan(step, h, (Ws, bs, gammas, betas))
    return _tc_out(h, W_out)
